# baseline (speedup 1.0000x reference)
import jax
import jax.numpy as jnp
from jax import lax
from jax.experimental import pallas as pl
from jax.experimental.pallas import tpu as pltpu
from jax.experimental.pallas import tpu_sc as plsc

_NC = 2
_NS = 16
_CH = 80
_DW = 128


_S = 3


def _edge_call(hm, ep, src, dst, N, E, G, nj):
    Ew = (E // _NS) if G == 2 else (E // (_NC * _NS))
    nchunks = Ew // _CH
    assert Ew % _CH == 0
    nsup, ntail = divmod(nchunks, _S)
    assert nsup > 0
    RSTART = (N // _NS) // 8 * 8
    RLEN = N - RSTART * (_NS - 1)
    assert RLEN % _CH == 0
    nz_full = RLEN // _CH

    def body(hm_r, ep_r, src_r, dst_r, agg_r, idx_s, idx_d, gbuf, ebuf, accum,
             sem_i, *sems):
        sem_g = sems[:_S]
        sem_s = sems[_S:]
        cid = lax.axis_index("c")
        sid = lax.axis_index("s")
        tsel = cid if G == 2 else 0
        ebase = 0 if G == 2 else cid * (E // _NC)
        ebase_h = 0 if G == 2 else cid * (E // (2 * _NC))
        zv = jnp.zeros((16,), jnp.float32)

        def zrow(r, carry):
            for j in range(_DW // 16):
                gbuf[0, r, pl.ds(j * 16, 16)] = zv
            return carry

        lax.fori_loop(0, _CH, zrow, 0)
        r0 = RSTART * sid
        for k in range(nz_full):
            pltpu.sync_copy(gbuf.at[0], accum.at[pl.ds(r0 + k * _CH, _CH)])
        plsc.subcore_barrier()

        hm_c = hm_r.at[tsel]
        ep_c = ep_r.at[tsel]

        def scatter_wait(j):
            pltpu.make_async_copy(gbuf.at[j], accum.at[idx_d.at[j]],
                                  sem_s[j]).wait()

        def sup(it, carry):
            base = ebase + sid * Ew + it * (_S * _CH)
            base2 = ebase_h + sid * (Ew // 2) + it * (_S * _CH // 2)

            @pl.when(it > 0)
            def _():
                for j in range(_S):
                    scatter_wait(j)

            for j in range(_S):
                pltpu.async_copy(src_r.at[pl.ds(base + j * _CH, _CH)],
                                 idx_s.at[j], sem_i)
                pltpu.async_copy(dst_r.at[pl.ds(base + j * _CH, _CH)],
                                 idx_d.at[j], sem_i)
            for j in range(_S):
                pltpu.make_async_copy(src_r.at[pl.ds(base + j * _CH, _CH)],
                                      idx_s.at[j], sem_i).wait()
                pltpu.make_async_copy(dst_r.at[pl.ds(base + j * _CH, _CH)],
                                      idx_d.at[j], sem_i).wait()
            for j in range(_S):
                pltpu.async_copy(hm_c.at[idx_s.at[j]], gbuf.at[j], sem_g[j])
                pltpu.async_copy(
                    ep_c.at[pl.ds(base2 + j * (_CH // 2), _CH // 2)],
                    ebuf.at[j], sem_g[j])
            for j in range(_S):
                pltpu.make_async_copy(hm_c.at[idx_s.at[j]], gbuf.at[j],
                                      sem_g[j]).wait()
                pltpu.make_async_copy(
                    ep_c.at[pl.ds(base2 + j * (_CH // 2), _CH // 2)],
                    ebuf.at[j], sem_g[j]).wait()

                def rbody(p, c2):
                    ra = 2 * p
                    rb = ra + 1
                    for q in range(nj):
                        sl = pl.ds(q * 16, 16)
                        ew = ebuf[j, p, sl]
                        ea = lax.bitcast_convert_type(ew << 16, jnp.float32)
                        eb2 = lax.bitcast_convert_type(ew & (-65536),
                                                       jnp.float32)
                        gbuf[j, ra, sl] = jnp.maximum(
                            gbuf[j, ra, sl] + ea, 0.0)
                        gbuf[j, rb, sl] = jnp.maximum(
                            gbuf[j, rb, sl] + eb2, 0.0)
                    return c2

                lax.fori_loop(0, _CH // 2, rbody, 0)
                pltpu.async_copy(gbuf.at[j], accum.at[idx_d.at[j]],
                                 sem_s[j], add=True)
            return carry

        lax.fori_loop(0, nsup, sup, 0)
        for t in range(ntail):
            c = nsup * _S + t
            eb = ebase + sid * Ew + c * _CH
            eb2 = ebase_h + sid * (Ew // 2) + c * (_CH // 2)
            scatter_wait(t)
            pltpu.sync_copy(src_r.at[pl.ds(eb, _CH)], idx_s.at[t])
            pltpu.sync_copy(dst_r.at[pl.ds(eb, _CH)], idx_d.at[t])
            pltpu.async_copy(hm_c.at[idx_s.at[t]], gbuf.at[t], sem_g[t])
            pltpu.async_copy(ep_c.at[pl.ds(eb2, _CH // 2)],
                             ebuf.at[t], sem_g[t])
            pltpu.make_async_copy(hm_c.at[idx_s.at[t]], gbuf.at[t],
                                  sem_g[t]).wait()
            pltpu.make_async_copy(ep_c.at[pl.ds(eb2, _CH // 2)],
                                  ebuf.at[t], sem_g[t]).wait()

            def tbody(p, c2):
                ra = 2 * p
                rb = ra + 1
                for q in range(nj):
                    sl = pl.ds(q * 16, 16)
                    ew = ebuf[t, p, sl]
                    ea = lax.bitcast_convert_type(ew << 16, jnp.float32)
                    eb2 = lax.bitcast_convert_type(ew & (-65536),
                                                   jnp.float32)
                    gbuf[t, ra, sl] = jnp.maximum(gbuf[t, ra, sl] + ea, 0.0)
                    gbuf[t, rb, sl] = jnp.maximum(gbuf[t, rb, sl] + eb2, 0.0)
                return c2

            lax.fori_loop(0, _CH // 2, tbody, 0)
            pltpu.async_copy(gbuf.at[t], accum.at[idx_d.at[t]],
                             sem_s[t], add=True)
        for j in range(_S):
            scatter_wait(j)
        plsc.subcore_barrier()
        pltpu.sync_copy(accum.at[pl.ds(r0, RLEN)],
                        agg_r.at[cid, pl.ds(r0, RLEN)])

    k = pl.kernel(
        body,
        out_type=jax.ShapeDtypeStruct((_NC, N, _DW), jnp.float32),
        mesh=plsc.VectorSubcoreMesh(core_axis_name="c", subcore_axis_name="s"),
        scratch_types=[
            pltpu.VMEM((_S, _CH), jnp.int32),
            pltpu.VMEM((_S, _CH), jnp.int32),
            pltpu.VMEM((_S, _CH, _DW), jnp.float32),
            pltpu.VMEM((_S, _CH // 2, _DW), jnp.int32),
            pltpu.VMEM_SHARED((N, _DW), jnp.float32),
        ] + [pltpu.SemaphoreType.DMA] * (1 + 2 * _S),
    )
    return k(hm, ep, src, dst)


def _eproj_call(edge_attr, wbots, gs):
    E, Da = edge_attr.shape
    BE = 2000
    steps = E // BE
    wcat = jnp.concatenate(wbots, axis=1)
    W = wcat.shape[1]
    outs_f = [w.shape[1] for w in wbots]

    def body(ea_ref, w_ref, *outs):
        res = jnp.dot(ea_ref[...], w_ref[...], preferred_element_type=jnp.float32)
        off = 0
        for li, (o, of, g) in enumerate(zip(outs, outs_f, gs)):
            for c in range(g):
                lo = off + c * _DW
                hi = min(off + of, lo + _DW)
                blk = res[:, lo:hi]
                if hi - lo < _DW:
                    pads = []
                    if li == 0:
                        pads.append(jnp.full((BE, 1), 1.0, jnp.float32))
                    fill = _DW - (hi - lo) - len(pads)
                    pads.append(jnp.zeros((BE, fill), jnp.float32))
                    blk = jnp.concatenate([blk] + pads, axis=1)
                o[c] = pltpu.bitcast(blk.astype(jnp.bfloat16), jnp.int32)
            off += of

    return pl.pallas_call(
        body,
        grid=(steps,),
        in_specs=[pl.BlockSpec((BE, Da), lambda i: (i, 0)),
                  pl.BlockSpec((Da, W), lambda i: (0, 0))],
        out_specs=[pl.BlockSpec((g, BE // 2, _DW), lambda i: (0, i, 0))
                   for g in gs],
        out_shape=[jax.ShapeDtypeStruct((g, E // 2, _DW), jnp.int32)
                   for g in gs],
    )(edge_attr, wcat)


def _hmsg_call(feat, wtop, G):
    N, in_f = feat.shape
    out_f = wtop.shape[1]
    BN = 2000
    steps = N // BN

    def body(f_ref, w_ref, o_ref):
        res = jnp.dot(f_ref[...], w_ref[...], preferred_element_type=jnp.float32)
        for c in range(G):
            lo = c * _DW
            hi = min(out_f, lo + _DW)
            blk = res[:, lo:hi]
            if hi - lo < _DW:
                blk = jnp.concatenate(
                    [blk, jnp.zeros((BN, _DW - (hi - lo)), jnp.float32)], axis=1)
            o_ref[c] = blk

    return pl.pallas_call(
        body,
        grid=(steps,),
        in_specs=[pl.BlockSpec((BN, in_f), lambda i: (i, 0)),
                  pl.BlockSpec((in_f, out_f), lambda i: (0, 0))],
        out_specs=pl.BlockSpec((G, BN, _DW), lambda i: (0, i, 0)),
        out_shape=jax.ShapeDtypeStruct((G, N, _DW), jnp.float32),
    )(feat, wtop)


def _post_call(feat, agg, rdeg, wself, b_row, wcls_row, G, deg_col=None):
    N, in_f = feat.shape
    out_f = wself.shape[1]
    BN = 2000
    steps = N // BN
    first = rdeg is None

    def body(*refs):
        if first:
            f_ref, a_ref, w_ref, b_ref, wc_ref, fo_ref, rd_ref, s_ref = refs
        else:
            f_ref, a_ref, rd_in, w_ref, b_ref, wc_ref, fo_ref, s_ref = refs
        i = pl.program_id(0)
        if G == 1:
            asum = a_ref[0] + a_ref[1]
            agg_b = asum[:, :out_f]
        else:
            agg_b = jnp.concatenate([a_ref[0], a_ref[1]], axis=1)
        if first:
            deg = a_ref[0, :, deg_col:deg_col + 1] + a_ref[1, :, deg_col:deg_col + 1]
            rd = 1.0 / jnp.maximum(deg, 1.0)
            rd_ref[...] = rd
        else:
            rd = rd_in[...]
        f = jnp.maximum(
            jnp.dot(f_ref[...], w_ref[...], preferred_element_type=jnp.float32)
            + agg_b * rd + b_ref[...], 0.0)
        fo_ref[...] = f
        ps = jnp.sum(f * wc_ref[...]).reshape(1, 1)

        @pl.when(i == 0)
        def _():
            s_ref[...] = jnp.zeros((1, 1), jnp.float32)

        s_ref[...] += ps

    in_specs = [pl.BlockSpec((BN, in_f), lambda i: (i, 0)),
                pl.BlockSpec((_NC, BN, _DW), lambda i: (0, i, 0))]
    operands = [feat, agg]
    if not first:
        in_specs.append(pl.BlockSpec((BN, 1), lambda i: (i, 0)))
        operands.append(rdeg)
    in_specs += [pl.BlockSpec((in_f, out_f), lambda i: (0, 0)),
                 pl.BlockSpec((1, out_f), lambda i: (0, 0)),
                 pl.BlockSpec((1, out_f), lambda i: (0, 0))]
    operands += [wself, b_row, wcls_row]
    out_specs = [pl.BlockSpec((BN, out_f), lambda i: (i, 0))]
    out_shape = [jax.ShapeDtypeStruct((N, out_f), jnp.float32)]
    if first:
        out_specs.append(pl.BlockSpec((BN, 1), lambda i: (i, 0)))
        out_shape.append(jax.ShapeDtypeStruct((N, 1), jnp.float32))
    out_specs.append(pl.BlockSpec((1, 1), lambda i: (0, 0)))
    out_shape.append(jax.ShapeDtypeStruct((1, 1), jnp.float32))

    return pl.pallas_call(
        body,
        grid=(steps,),
        in_specs=in_specs,
        out_specs=out_specs,
        out_shape=out_shape,
    )(*operands)


def kernel(x, edge_index, edge_attr, params):
    N = x.shape[0]
    E = edge_index.shape[1]
    src = edge_index[0].astype(jnp.int32)
    dst = edge_index[1].astype(jnp.int32)

    in_fs = [p[1].shape[0] for p in params]
    out_fs = [p[0].shape[1] for p in params]
    gs = [-(-of // _DW) for of in out_fs]
    assert out_fs[0] < _DW, "layer-0 padding must have room for the degree column"
    wtops = [p[0][:f] for p, f in zip(params, in_fs)]
    wbots = [p[0][f:] for p, f in zip(params, in_fs)]

    eps = _eproj_call(edge_attr, wbots, gs)

    feat = x
    rdeg = None
    score = None
    for l, (W_msg, W_self, b, W_cls, b_cls) in enumerate(params):
        hm = _hmsg_call(feat, wtops[l], gs[l])
        nj = -(-(out_fs[l] + (1 if l == 0 else 0)) // 16) if gs[l] == 1 else _DW // 16
        agg = _edge_call(hm, eps[l], src, dst, N, E, gs[l], nj)
        outs = _post_call(feat, agg, rdeg, W_self, b.reshape(1, -1),
                          jnp.transpose(W_cls), gs[l],
                          deg_col=(out_fs[0] if l == 0 else None))
        if l == 0:
            feat, rdeg, s = outs
        else:
            feat, s = outs
        contrib = s / N + b_cls.reshape(1, 1)
        score = contrib if score is None else score + contrib
    return score


# speedup vs baseline: 1.1893x; 1.1893x over previous
import jax
import jax.numpy as jnp
from jax import lax
from jax.experimental import pallas as pl
from jax.experimental.pallas import tpu as pltpu
from jax.experimental.pallas import tpu_sc as plsc

_NC = 2
_NS = 16
_CH = 80
_DW = 128


_S = 2


def _edge_call(hm, ep, src, dst, N, E, G, nj):
    Ew = (E // _NS) if G == 2 else (E // (_NC * _NS))
    nchunks = Ew // _CH
    assert Ew % _CH == 0
    nsup, ntail = divmod(nchunks, _S)
    assert nsup > 0
    RSTART = (N // _NS) // 8 * 8
    RLEN = N - RSTART * (_NS - 1)
    assert RLEN % _CH == 0
    nz_full = RLEN // _CH

    def body(hm_r, ep_r, src_r, dst_r, agg_r, idx_s, idx_d, gbuf, ebuf, accum,
             sem_i, *sems):
        sem_g = sems[:_S]
        sem_s = sems[_S:]
        cid = lax.axis_index("c")
        sid = lax.axis_index("s")
        tsel = cid if G == 2 else 0
        ebase = 0 if G == 2 else cid * (E // _NC)
        zv = jnp.zeros((16,), jnp.float32)

        def zrow(r, carry):
            for j in range(_DW // 16):
                gbuf[0, r, pl.ds(j * 16, 16)] = zv
            return carry

        lax.fori_loop(0, _CH, zrow, 0)
        r0 = RSTART * sid
        for k in range(nz_full):
            pltpu.sync_copy(gbuf.at[0], accum.at[pl.ds(r0 + k * _CH, _CH)])
        plsc.subcore_barrier()

        hm_c = hm_r.at[tsel]
        ep_c = ep_r.at[tsel]

        def scatter_wait(j):
            pltpu.make_async_copy(gbuf.at[j], accum.at[idx_d.at[j]],
                                  sem_s[j]).wait()

        def sup(it, carry):
            base = ebase + sid * Ew + it * (_S * _CH)

            @pl.when(it > 0)
            def _():
                for j in range(_S):
                    scatter_wait(j)

            for j in range(_S):
                pltpu.async_copy(src_r.at[pl.ds(base + j * _CH, _CH)],
                                 idx_s.at[j], sem_i)
                pltpu.async_copy(dst_r.at[pl.ds(base + j * _CH, _CH)],
                                 idx_d.at[j], sem_i)
            for j in range(_S):
                pltpu.make_async_copy(src_r.at[pl.ds(base + j * _CH, _CH)],
                                      idx_s.at[j], sem_i).wait()
                pltpu.make_async_copy(dst_r.at[pl.ds(base + j * _CH, _CH)],
                                      idx_d.at[j], sem_i).wait()
            for j in range(_S):
                pltpu.async_copy(hm_c.at[idx_s.at[j]], gbuf.at[j], sem_g[j])
                pltpu.async_copy(ep_c.at[pl.ds(base + j * _CH, _CH)],
                                 ebuf.at[j], sem_g[j])
            for j in range(_S):
                pltpu.make_async_copy(hm_c.at[idx_s.at[j]], gbuf.at[j],
                                      sem_g[j]).wait()
                pltpu.make_async_copy(ep_c.at[pl.ds(base + j * _CH, _CH)],
                                      ebuf.at[j], sem_g[j]).wait()

                @plsc.parallel_loop(0, _CH, unroll=4)
                def _(r):
                    for q in range(nj):
                        sl = pl.ds(q * 16, 16)
                        gbuf[j, r, sl] = jnp.maximum(
                            gbuf[j, r, sl] + ebuf[j, r, sl], 0.0)
                pltpu.async_copy(gbuf.at[j], accum.at[idx_d.at[j]],
                                 sem_s[j], add=True)
            return carry

        lax.fori_loop(0, nsup, sup, 0)
        for t in range(ntail):
            c = nsup * _S + t
            eb = ebase + sid * Ew + c * _CH
            scatter_wait(t)
            pltpu.sync_copy(src_r.at[pl.ds(eb, _CH)], idx_s.at[t])
            pltpu.sync_copy(dst_r.at[pl.ds(eb, _CH)], idx_d.at[t])
            pltpu.async_copy(hm_c.at[idx_s.at[t]], gbuf.at[t], sem_g[t])
            pltpu.async_copy(ep_c.at[pl.ds(eb, _CH)], ebuf.at[t], sem_g[t])
            pltpu.make_async_copy(hm_c.at[idx_s.at[t]], gbuf.at[t],
                                  sem_g[t]).wait()
            pltpu.make_async_copy(ep_c.at[pl.ds(eb, _CH)], ebuf.at[t],
                                  sem_g[t]).wait()

            @plsc.parallel_loop(0, _CH, unroll=4)
            def _(r):
                for q in range(nj):
                    sl = pl.ds(q * 16, 16)
                    gbuf[t, r, sl] = jnp.maximum(
                        gbuf[t, r, sl] + ebuf[t, r, sl], 0.0)
            pltpu.async_copy(gbuf.at[t], accum.at[idx_d.at[t]],
                             sem_s[t], add=True)
        for j in range(_S):
            scatter_wait(j)
        plsc.subcore_barrier()
        pltpu.sync_copy(accum.at[pl.ds(r0, RLEN)],
                        agg_r.at[cid, pl.ds(r0, RLEN)])

    k = pl.kernel(
        body,
        out_type=jax.ShapeDtypeStruct((_NC, N, _DW), jnp.float32),
        mesh=plsc.VectorSubcoreMesh(core_axis_name="c", subcore_axis_name="s"),
        scratch_types=[
            pltpu.VMEM((_S, _CH), jnp.int32),
            pltpu.VMEM((_S, _CH), jnp.int32),
            pltpu.VMEM((_S, _CH, _DW), jnp.float32),
            pltpu.VMEM((_S, _CH, _DW), jnp.float32),
            pltpu.VMEM_SHARED((N, _DW), jnp.float32),
        ] + [pltpu.SemaphoreType.DMA] * (1 + 2 * _S),
    )
    return k(hm, ep, src, dst)


def _eproj_call(edge_attr, wbots, gs):
    E, Da = edge_attr.shape
    BE = 2000
    steps = E // BE
    wcat = jnp.concatenate(wbots, axis=1)
    W = wcat.shape[1]
    outs_f = [w.shape[1] for w in wbots]

    def body(ea_ref, w_ref, *outs):
        res = jnp.dot(ea_ref[...], w_ref[...], preferred_element_type=jnp.float32)
        off = 0
        for li, (o, of, g) in enumerate(zip(outs, outs_f, gs)):
            for c in range(g):
                lo = off + c * _DW
                hi = min(off + of, lo + _DW)
                blk = res[:, lo:hi]
                if hi - lo < _DW:
                    pads = []
                    if li == 0:
                        pads.append(jnp.full((BE, 1), 1.0, jnp.float32))
                    fill = _DW - (hi - lo) - len(pads)
                    pads.append(jnp.zeros((BE, fill), jnp.float32))
                    blk = jnp.concatenate([blk] + pads, axis=1)
                o[c] = blk
            off += of

    return pl.pallas_call(
        body,
        grid=(steps,),
        in_specs=[pl.BlockSpec((BE, Da), lambda i: (i, 0)),
                  pl.BlockSpec((Da, W), lambda i: (0, 0))],
        out_specs=[pl.BlockSpec((g, BE, _DW), lambda i: (0, i, 0)) for g in gs],
        out_shape=[jax.ShapeDtypeStruct((g, E, _DW), jnp.float32) for g in gs],
    )(edge_attr, wcat)


def _hmsg_call(feat, wtop, G):
    N, in_f = feat.shape
    out_f = wtop.shape[1]
    BN = 2000
    steps = N // BN

    def body(f_ref, w_ref, o_ref):
        res = jnp.dot(f_ref[...], w_ref[...], preferred_element_type=jnp.float32)
        for c in range(G):
            lo = c * _DW
            hi = min(out_f, lo + _DW)
            blk = res[:, lo:hi]
            if hi - lo < _DW:
                blk = jnp.concatenate(
                    [blk, jnp.zeros((BN, _DW - (hi - lo)), jnp.float32)], axis=1)
            o_ref[c] = blk

    return pl.pallas_call(
        body,
        grid=(steps,),
        in_specs=[pl.BlockSpec((BN, in_f), lambda i: (i, 0)),
                  pl.BlockSpec((in_f, out_f), lambda i: (0, 0))],
        out_specs=pl.BlockSpec((G, BN, _DW), lambda i: (0, i, 0)),
        out_shape=jax.ShapeDtypeStruct((G, N, _DW), jnp.float32),
    )(feat, wtop)


def _post_call(feat, agg, rdeg, wself, b_row, wcls_row, G, deg_col=None):
    N, in_f = feat.shape
    out_f = wself.shape[1]
    BN = 2000
    steps = N // BN
    first = rdeg is None

    def body(*refs):
        if first:
            f_ref, a_ref, w_ref, b_ref, wc_ref, fo_ref, rd_ref, s_ref = refs
        else:
            f_ref, a_ref, rd_in, w_ref, b_ref, wc_ref, fo_ref, s_ref = refs
        i = pl.program_id(0)
        if G == 1:
            asum = a_ref[0] + a_ref[1]
            agg_b = asum[:, :out_f]
        else:
            agg_b = jnp.concatenate([a_ref[0], a_ref[1]], axis=1)
        if first:
            deg = a_ref[0, :, deg_col:deg_col + 1] + a_ref[1, :, deg_col:deg_col + 1]
            rd = 1.0 / jnp.maximum(deg, 1.0)
            rd_ref[...] = rd
        else:
            rd = rd_in[...]
        f = jnp.maximum(
            jnp.dot(f_ref[...], w_ref[...], preferred_element_type=jnp.float32)
            + agg_b * rd + b_ref[...], 0.0)
        fo_ref[...] = f
        ps = jnp.sum(f * wc_ref[...]).reshape(1, 1)

        @pl.when(i == 0)
        def _():
            s_ref[...] = jnp.zeros((1, 1), jnp.float32)

        s_ref[...] += ps

    in_specs = [pl.BlockSpec((BN, in_f), lambda i: (i, 0)),
                pl.BlockSpec((_NC, BN, _DW), lambda i: (0, i, 0))]
    operands = [feat, agg]
    if not first:
        in_specs.append(pl.BlockSpec((BN, 1), lambda i: (i, 0)))
        operands.append(rdeg)
    in_specs += [pl.BlockSpec((in_f, out_f), lambda i: (0, 0)),
                 pl.BlockSpec((1, out_f), lambda i: (0, 0)),
                 pl.BlockSpec((1, out_f), lambda i: (0, 0))]
    operands += [wself, b_row, wcls_row]
    out_specs = [pl.BlockSpec((BN, out_f), lambda i: (i, 0))]
    out_shape = [jax.ShapeDtypeStruct((N, out_f), jnp.float32)]
    if first:
        out_specs.append(pl.BlockSpec((BN, 1), lambda i: (i, 0)))
        out_shape.append(jax.ShapeDtypeStruct((N, 1), jnp.float32))
    out_specs.append(pl.BlockSpec((1, 1), lambda i: (0, 0)))
    out_shape.append(jax.ShapeDtypeStruct((1, 1), jnp.float32))

    return pl.pallas_call(
        body,
        grid=(steps,),
        in_specs=in_specs,
        out_specs=out_specs,
        out_shape=out_shape,
    )(*operands)


def kernel(x, edge_index, edge_attr, params):
    N = x.shape[0]
    E = edge_index.shape[1]
    src = edge_index[0].astype(jnp.int32)
    dst = edge_index[1].astype(jnp.int32)

    in_fs = [p[1].shape[0] for p in params]
    out_fs = [p[0].shape[1] for p in params]
    gs = [-(-of // _DW) for of in out_fs]
    assert out_fs[0] < _DW, "layer-0 padding must have room for the degree column"
    wtops = [p[0][:f] for p, f in zip(params, in_fs)]
    wbots = [p[0][f:] for p, f in zip(params, in_fs)]

    eps = _eproj_call(edge_attr, wbots, gs)

    feat = x
    rdeg = None
    score = None
    for l, (W_msg, W_self, b, W_cls, b_cls) in enumerate(params):
        hm = _hmsg_call(feat, wtops[l], gs[l])
        nj = -(-(out_fs[l] + (1 if l == 0 else 0)) // 16) if gs[l] == 1 else _DW // 16
        agg = _edge_call(hm, eps[l], src, dst, N, E, gs[l], nj)
        outs = _post_call(feat, agg, rdeg, W_self, b.reshape(1, -1),
                          jnp.transpose(W_cls), gs[l],
                          deg_col=(out_fs[0] if l == 0 else None))
        if l == 0:
            feat, rdeg, s = outs
        else:
            feat, s = outs
        contrib = s / N + b_cls.reshape(1, 1)
        score = contrib if score is None else score + contrib
    return score


# speedup vs baseline: 1.4001x; 1.1772x over previous
import jax
import jax.numpy as jnp
from jax import lax
from jax.experimental import pallas as pl
from jax.experimental.pallas import tpu as pltpu
from jax.experimental.pallas import tpu_sc as plsc

_NC = 2
_NS = 16
_CH = 80
_DW = 128


_S = 2


def _edge_call(hm, ep, src, dst, N, E, G, nj):
    Ew = (E // _NS) if G == 2 else (E // (_NC * _NS))
    nchunks = Ew // _CH
    assert Ew % _CH == 0
    nsup, ntail = divmod(nchunks, _S)
    assert nsup > 0
    RSTART = (N // _NS) // 8 * 8
    RLEN = N - RSTART * (_NS - 1)
    assert RLEN % _CH == 0
    nz_full = RLEN // _CH

    def body(hm_r, ep_r, src_r, dst_r, agg_r, idx_s, idx_d, gbuf, ebuf, accum,
             sem_i, *sems):
        sem_g = sems[:_S]
        sem_s = sems[_S:]
        cid = lax.axis_index("c")
        sid = lax.axis_index("s")
        tsel = cid if G == 2 else 0
        ebase = 0 if G == 2 else cid * (E // _NC)
        ebase_h = 0 if G == 2 else cid * (E // (2 * _NC))
        zv = jnp.zeros((16,), jnp.float32)

        def zrow(r, carry):
            for j in range(_DW // 16):
                gbuf[0, r, pl.ds(j * 16, 16)] = zv
            return carry

        lax.fori_loop(0, _CH, zrow, 0)
        r0 = RSTART * sid
        for k in range(nz_full):
            pltpu.sync_copy(gbuf.at[0], accum.at[pl.ds(r0 + k * _CH, _CH)])
        plsc.subcore_barrier()

        hm_c = hm_r.at[tsel]
        ep_c = ep_r.at[tsel]

        def scatter_wait(j):
            pltpu.make_async_copy(gbuf.at[j], accum.at[idx_d.at[j]],
                                  sem_s[j]).wait()

        def sup(it, carry):
            base = ebase + sid * Ew + it * (_S * _CH)
            base2 = ebase_h + sid * (Ew // 2) + it * (_S * _CH // 2)

            @pl.when(it > 0)
            def _():
                for j in range(_S):
                    scatter_wait(j)

            for j in range(_S):
                pltpu.async_copy(src_r.at[pl.ds(base + j * _CH, _CH)],
                                 idx_s.at[j], sem_i)
                pltpu.async_copy(dst_r.at[pl.ds(base + j * _CH, _CH)],
                                 idx_d.at[j], sem_i)
            for j in range(_S):
                pltpu.make_async_copy(src_r.at[pl.ds(base + j * _CH, _CH)],
                                      idx_s.at[j], sem_i).wait()
                pltpu.make_async_copy(dst_r.at[pl.ds(base + j * _CH, _CH)],
                                      idx_d.at[j], sem_i).wait()
            for j in range(_S):
                pltpu.async_copy(hm_c.at[idx_s.at[j]], gbuf.at[j], sem_g[j])
                pltpu.async_copy(
                    ep_c.at[pl.ds(base2 + j * (_CH // 2), _CH // 2)],
                    ebuf.at[j], sem_g[j])
            for j in range(_S):
                pltpu.make_async_copy(hm_c.at[idx_s.at[j]], gbuf.at[j],
                                      sem_g[j]).wait()
                pltpu.make_async_copy(
                    ep_c.at[pl.ds(base2 + j * (_CH // 2), _CH // 2)],
                    ebuf.at[j], sem_g[j]).wait()

                @plsc.parallel_loop(0, _CH // 2, unroll=2)
                def _(p):
                    ra = 2 * p
                    rb = ra + 1
                    for q in range(nj):
                        sl = pl.ds(q * 16, 16)
                        ew = ebuf[j, p, sl]
                        ea = lax.bitcast_convert_type(ew << 16, jnp.float32)
                        eb3 = lax.bitcast_convert_type(ew & (-65536),
                                                       jnp.float32)
                        gbuf[j, ra, sl] = jnp.maximum(
                            gbuf[j, ra, sl] + ea, 0.0)
                        gbuf[j, rb, sl] = jnp.maximum(
                            gbuf[j, rb, sl] + eb3, 0.0)
                pltpu.async_copy(gbuf.at[j], accum.at[idx_d.at[j]],
                                 sem_s[j], add=True)
            return carry

        lax.fori_loop(0, nsup, sup, 0)
        for t in range(ntail):
            c = nsup * _S + t
            eb = ebase + sid * Ew + c * _CH
            eb2 = ebase_h + sid * (Ew // 2) + c * (_CH // 2)
            scatter_wait(t)
            pltpu.sync_copy(src_r.at[pl.ds(eb, _CH)], idx_s.at[t])
            pltpu.sync_copy(dst_r.at[pl.ds(eb, _CH)], idx_d.at[t])
            pltpu.async_copy(hm_c.at[idx_s.at[t]], gbuf.at[t], sem_g[t])
            pltpu.async_copy(ep_c.at[pl.ds(eb2, _CH // 2)],
                             ebuf.at[t], sem_g[t])
            pltpu.make_async_copy(hm_c.at[idx_s.at[t]], gbuf.at[t],
                                  sem_g[t]).wait()
            pltpu.make_async_copy(ep_c.at[pl.ds(eb2, _CH // 2)],
                                  ebuf.at[t], sem_g[t]).wait()

            @plsc.parallel_loop(0, _CH // 2, unroll=2)
            def _(p):
                ra = 2 * p
                rb = ra + 1
                for q in range(nj):
                    sl = pl.ds(q * 16, 16)
                    ew = ebuf[t, p, sl]
                    ea = lax.bitcast_convert_type(ew << 16, jnp.float32)
                    eb3 = lax.bitcast_convert_type(ew & (-65536),
                                                   jnp.float32)
                    gbuf[t, ra, sl] = jnp.maximum(gbuf[t, ra, sl] + ea, 0.0)
                    gbuf[t, rb, sl] = jnp.maximum(gbuf[t, rb, sl] + eb3, 0.0)
            pltpu.async_copy(gbuf.at[t], accum.at[idx_d.at[t]],
                             sem_s[t], add=True)
        for j in range(_S):
            scatter_wait(j)
        plsc.subcore_barrier()
        pltpu.sync_copy(accum.at[pl.ds(r0, RLEN)],
                        agg_r.at[cid, pl.ds(r0, RLEN)])

    k = pl.kernel(
        body,
        out_type=jax.ShapeDtypeStruct((_NC, N, _DW), jnp.float32),
        mesh=plsc.VectorSubcoreMesh(core_axis_name="c", subcore_axis_name="s"),
        scratch_types=[
            pltpu.VMEM((_S, _CH), jnp.int32),
            pltpu.VMEM((_S, _CH), jnp.int32),
            pltpu.VMEM((_S, _CH, _DW), jnp.float32),
            pltpu.VMEM((_S, _CH // 2, _DW), jnp.int32),
            pltpu.VMEM_SHARED((N, _DW), jnp.float32),
        ] + [pltpu.SemaphoreType.DMA] * (1 + 2 * _S),
    )
    return k(hm, ep, src, dst)


def _eproj_call(edge_attr, wbots, gs):
    E, Da = edge_attr.shape
    BE = 2000
    steps = E // BE
    wcat = jnp.concatenate(wbots, axis=1)
    W = wcat.shape[1]
    outs_f = [w.shape[1] for w in wbots]

    def body(ea_ref, w_ref, *outs):
        res = jnp.dot(ea_ref[...], w_ref[...], preferred_element_type=jnp.float32)
        off = 0
        for li, (o, of, g) in enumerate(zip(outs, outs_f, gs)):
            for c in range(g):
                lo = off + c * _DW
                hi = min(off + of, lo + _DW)
                blk = res[:, lo:hi]
                if hi - lo < _DW:
                    pads = []
                    if li == 0:
                        pads.append(jnp.full((BE, 1), 1.0, jnp.float32))
                    fill = _DW - (hi - lo) - len(pads)
                    pads.append(jnp.zeros((BE, fill), jnp.float32))
                    blk = jnp.concatenate([blk] + pads, axis=1)
                o[c] = pltpu.bitcast(blk.astype(jnp.bfloat16), jnp.int32)
            off += of

    return pl.pallas_call(
        body,
        grid=(steps,),
        in_specs=[pl.BlockSpec((BE, Da), lambda i: (i, 0)),
                  pl.BlockSpec((Da, W), lambda i: (0, 0))],
        out_specs=[pl.BlockSpec((g, BE // 2, _DW), lambda i: (0, i, 0))
                   for g in gs],
        out_shape=[jax.ShapeDtypeStruct((g, E // 2, _DW), jnp.int32)
                   for g in gs],
    )(edge_attr, wcat)


def _hmsg_call(feat, wtop, G):
    N, in_f = feat.shape
    out_f = wtop.shape[1]
    BN = 2000
    steps = N // BN

    def body(f_ref, w_ref, o_ref):
        res = jnp.dot(f_ref[...], w_ref[...], preferred_element_type=jnp.float32)
        for c in range(G):
            lo = c * _DW
            hi = min(out_f, lo + _DW)
            blk = res[:, lo:hi]
            if hi - lo < _DW:
                blk = jnp.concatenate(
                    [blk, jnp.zeros((BN, _DW - (hi - lo)), jnp.float32)], axis=1)
            o_ref[c] = blk

    return pl.pallas_call(
        body,
        grid=(steps,),
        in_specs=[pl.BlockSpec((BN, in_f), lambda i: (i, 0)),
                  pl.BlockSpec((in_f, out_f), lambda i: (0, 0))],
        out_specs=pl.BlockSpec((G, BN, _DW), lambda i: (0, i, 0)),
        out_shape=jax.ShapeDtypeStruct((G, N, _DW), jnp.float32),
    )(feat, wtop)


def _post_call(feat, agg, rdeg, wself, b_row, wcls_row, G, deg_col=None):
    N, in_f = feat.shape
    out_f = wself.shape[1]
    BN = 2000
    steps = N // BN
    first = rdeg is None

    def body(*refs):
        if first:
            f_ref, a_ref, w_ref, b_ref, wc_ref, fo_ref, rd_ref, s_ref = refs
        else:
            f_ref, a_ref, rd_in, w_ref, b_ref, wc_ref, fo_ref, s_ref = refs
        i = pl.program_id(0)
        if G == 1:
            asum = a_ref[0] + a_ref[1]
            agg_b = asum[:, :out_f]
        else:
            agg_b = jnp.concatenate([a_ref[0], a_ref[1]], axis=1)
        if first:
            deg = a_ref[0, :, deg_col:deg_col + 1] + a_ref[1, :, deg_col:deg_col + 1]
            rd = 1.0 / jnp.maximum(deg, 1.0)
            rd_ref[...] = rd
        else:
            rd = rd_in[...]
        f = jnp.maximum(
            jnp.dot(f_ref[...], w_ref[...], preferred_element_type=jnp.float32)
            + agg_b * rd + b_ref[...], 0.0)
        fo_ref[...] = f
        ps = jnp.sum(f * wc_ref[...]).reshape(1, 1)

        @pl.when(i == 0)
        def _():
            s_ref[...] = jnp.zeros((1, 1), jnp.float32)

        s_ref[...] += ps

    in_specs = [pl.BlockSpec((BN, in_f), lambda i: (i, 0)),
                pl.BlockSpec((_NC, BN, _DW), lambda i: (0, i, 0))]
    operands = [feat, agg]
    if not first:
        in_specs.append(pl.BlockSpec((BN, 1), lambda i: (i, 0)))
        operands.append(rdeg)
    in_specs += [pl.BlockSpec((in_f, out_f), lambda i: (0, 0)),
                 pl.BlockSpec((1, out_f), lambda i: (0, 0)),
                 pl.BlockSpec((1, out_f), lambda i: (0, 0))]
    operands += [wself, b_row, wcls_row]
    out_specs = [pl.BlockSpec((BN, out_f), lambda i: (i, 0))]
    out_shape = [jax.ShapeDtypeStruct((N, out_f), jnp.float32)]
    if first:
        out_specs.append(pl.BlockSpec((BN, 1), lambda i: (i, 0)))
        out_shape.append(jax.ShapeDtypeStruct((N, 1), jnp.float32))
    out_specs.append(pl.BlockSpec((1, 1), lambda i: (0, 0)))
    out_shape.append(jax.ShapeDtypeStruct((1, 1), jnp.float32))

    return pl.pallas_call(
        body,
        grid=(steps,),
        in_specs=in_specs,
        out_specs=out_specs,
        out_shape=out_shape,
    )(*operands)


def kernel(x, edge_index, edge_attr, params):
    N = x.shape[0]
    E = edge_index.shape[1]
    src = edge_index[0].astype(jnp.int32)
    dst = edge_index[1].astype(jnp.int32)

    in_fs = [p[1].shape[0] for p in params]
    out_fs = [p[0].shape[1] for p in params]
    gs = [-(-of // _DW) for of in out_fs]
    assert out_fs[0] < _DW, "layer-0 padding must have room for the degree column"
    wtops = [p[0][:f] for p, f in zip(params, in_fs)]
    wbots = [p[0][f:] for p, f in zip(params, in_fs)]

    eps = _eproj_call(edge_attr, wbots, gs)

    feat = x
    rdeg = None
    score = None
    for l, (W_msg, W_self, b, W_cls, b_cls) in enumerate(params):
        hm = _hmsg_call(feat, wtops[l], gs[l])
        nj = -(-(out_fs[l] + (1 if l == 0 else 0)) // 16) if gs[l] == 1 else _DW // 16
        agg = _edge_call(hm, eps[l], src, dst, N, E, gs[l], nj)
        outs = _post_call(feat, agg, rdeg, W_self, b.reshape(1, -1),
                          jnp.transpose(W_cls), gs[l],
                          deg_col=(out_fs[0] if l == 0 else None))
        if l == 0:
            feat, rdeg, s = outs
        else:
            feat, s = outs
        contrib = s / N + b_cls.reshape(1, 1)
        score = contrib if score is None else score + contrib
    return score


# speedup vs baseline: 1.5009x; 1.0720x over previous
import jax
import jax.numpy as jnp
from jax import lax
from jax.experimental import pallas as pl
from jax.experimental.pallas import tpu as pltpu
from jax.experimental.pallas import tpu_sc as plsc

_NC = 2
_NS = 16
_CH = 80
_DW = 128


_S = 3


def _edge_call(hm, ep, src, dst, N, E, G, nj):
    Ew = (E // _NS) if G == 2 else (E // (_NC * _NS))
    nchunks = Ew // _CH
    assert Ew % _CH == 0
    nsup, ntail = divmod(nchunks, _S)
    assert nsup > 0
    RSTART = (N // _NS) // 8 * 8
    RLEN = N - RSTART * (_NS - 1)
    assert RLEN % _CH == 0
    nz_full = RLEN // _CH

    def body(hm_r, ep_r, src_r, dst_r, agg_r, idx_s, idx_d, gbuf, ebuf, accum,
             sem_i, *sems):
        sem_g = sems[:_S]
        sem_s = sems[_S:]
        cid = lax.axis_index("c")
        sid = lax.axis_index("s")
        tsel = cid if G == 2 else 0
        ebase = 0 if G == 2 else cid * (E // _NC)
        ebase_h = 0 if G == 2 else cid * (E // (2 * _NC))
        zv = jnp.zeros((16,), jnp.float32)

        def zrow(r, carry):
            for j in range(_DW // 16):
                gbuf[0, r, pl.ds(j * 16, 16)] = zv
            return carry

        lax.fori_loop(0, _CH, zrow, 0)
        r0 = RSTART * sid
        for k in range(nz_full):
            pltpu.sync_copy(gbuf.at[0], accum.at[pl.ds(r0 + k * _CH, _CH)])
        plsc.subcore_barrier()

        hm_c = hm_r.at[tsel]
        ep_c = ep_r.at[tsel]

        def scatter_wait(j):
            pltpu.make_async_copy(gbuf.at[j], accum.at[idx_d.at[j]],
                                  sem_s[j]).wait()

        def sup(it, carry):
            base = ebase + sid * Ew + it * (_S * _CH)
            base2 = ebase_h + sid * (Ew // 2) + it * (_S * _CH // 2)

            @pl.when(it > 0)
            def _():
                for j in range(_S):
                    scatter_wait(j)

            for j in range(_S):
                pltpu.async_copy(src_r.at[pl.ds(base + j * _CH, _CH)],
                                 idx_s.at[j], sem_i)
                pltpu.async_copy(dst_r.at[pl.ds(base + j * _CH, _CH)],
                                 idx_d.at[j], sem_i)
            for j in range(_S):
                pltpu.make_async_copy(src_r.at[pl.ds(base + j * _CH, _CH)],
                                      idx_s.at[j], sem_i).wait()
                pltpu.make_async_copy(dst_r.at[pl.ds(base + j * _CH, _CH)],
                                      idx_d.at[j], sem_i).wait()
            for j in range(_S):
                pltpu.async_copy(hm_c.at[idx_s.at[j]], gbuf.at[j], sem_g[j])
                pltpu.async_copy(
                    ep_c.at[pl.ds(base2 + j * (_CH // 2), _CH // 2)],
                    ebuf.at[j], sem_g[j])
            for j in range(_S):
                pltpu.make_async_copy(hm_c.at[idx_s.at[j]], gbuf.at[j],
                                      sem_g[j]).wait()
                pltpu.make_async_copy(
                    ep_c.at[pl.ds(base2 + j * (_CH // 2), _CH // 2)],
                    ebuf.at[j], sem_g[j]).wait()

                @plsc.parallel_loop(0, _CH // 2, unroll=2)
                def _(p):
                    ra = 2 * p
                    rb = ra + 1
                    for q in range(nj):
                        sl = pl.ds(q * 16, 16)
                        ew = ebuf[j, p, sl]
                        ea = lax.bitcast_convert_type(ew << 16, jnp.float32)
                        eb3 = lax.bitcast_convert_type(ew & (-65536),
                                                       jnp.float32)
                        gbuf[j, ra, sl] = jnp.maximum(
                            gbuf[j, ra, sl] + ea, 0.0)
                        gbuf[j, rb, sl] = jnp.maximum(
                            gbuf[j, rb, sl] + eb3, 0.0)
                pltpu.async_copy(gbuf.at[j], accum.at[idx_d.at[j]],
                                 sem_s[j], add=True)
            return carry

        lax.fori_loop(0, nsup, sup, 0)
        for t in range(ntail):
            c = nsup * _S + t
            eb = ebase + sid * Ew + c * _CH
            eb2 = ebase_h + sid * (Ew // 2) + c * (_CH // 2)
            scatter_wait(t)
            pltpu.sync_copy(src_r.at[pl.ds(eb, _CH)], idx_s.at[t])
            pltpu.sync_copy(dst_r.at[pl.ds(eb, _CH)], idx_d.at[t])
            pltpu.async_copy(hm_c.at[idx_s.at[t]], gbuf.at[t], sem_g[t])
            pltpu.async_copy(ep_c.at[pl.ds(eb2, _CH // 2)],
                             ebuf.at[t], sem_g[t])
            pltpu.make_async_copy(hm_c.at[idx_s.at[t]], gbuf.at[t],
                                  sem_g[t]).wait()
            pltpu.make_async_copy(ep_c.at[pl.ds(eb2, _CH // 2)],
                                  ebuf.at[t], sem_g[t]).wait()

            @plsc.parallel_loop(0, _CH // 2, unroll=2)
            def _(p):
                ra = 2 * p
                rb = ra + 1
                for q in range(nj):
                    sl = pl.ds(q * 16, 16)
                    ew = ebuf[t, p, sl]
                    ea = lax.bitcast_convert_type(ew << 16, jnp.float32)
                    eb3 = lax.bitcast_convert_type(ew & (-65536),
                                                   jnp.float32)
                    gbuf[t, ra, sl] = jnp.maximum(gbuf[t, ra, sl] + ea, 0.0)
                    gbuf[t, rb, sl] = jnp.maximum(gbuf[t, rb, sl] + eb3, 0.0)
            pltpu.async_copy(gbuf.at[t], accum.at[idx_d.at[t]],
                             sem_s[t], add=True)
        for j in range(_S):
            scatter_wait(j)
        plsc.subcore_barrier()
        pltpu.sync_copy(accum.at[pl.ds(r0, RLEN)],
                        agg_r.at[cid, pl.ds(r0, RLEN)])

    k = pl.kernel(
        body,
        out_type=jax.ShapeDtypeStruct((_NC, N, _DW), jnp.float32),
        mesh=plsc.VectorSubcoreMesh(core_axis_name="c", subcore_axis_name="s"),
        scratch_types=[
            pltpu.VMEM((_S, _CH), jnp.int32),
            pltpu.VMEM((_S, _CH), jnp.int32),
            pltpu.VMEM((_S, _CH, _DW), jnp.float32),
            pltpu.VMEM((_S, _CH // 2, _DW), jnp.int32),
            pltpu.VMEM_SHARED((N, _DW), jnp.float32),
        ] + [pltpu.SemaphoreType.DMA] * (1 + 2 * _S),
    )
    return k(hm, ep, src, dst)


def _eproj_call(edge_attr, wbots, gs):
    E, Da = edge_attr.shape
    BE = 2000
    steps = E // BE
    wcat = jnp.concatenate(wbots, axis=1)
    W = wcat.shape[1]
    outs_f = [w.shape[1] for w in wbots]

    def body(ea_ref, w_ref, *outs):
        res = jnp.dot(ea_ref[...], w_ref[...], preferred_element_type=jnp.float32)
        off = 0
        for li, (o, of, g) in enumerate(zip(outs, outs_f, gs)):
            for c in range(g):
                lo = off + c * _DW
                hi = min(off + of, lo + _DW)
                blk = res[:, lo:hi]
                if hi - lo < _DW:
                    pads = []
                    if li == 0:
                        pads.append(jnp.full((BE, 1), 1.0, jnp.float32))
                    fill = _DW - (hi - lo) - len(pads)
                    pads.append(jnp.zeros((BE, fill), jnp.float32))
                    blk = jnp.concatenate([blk] + pads, axis=1)
                o[c] = pltpu.bitcast(blk.astype(jnp.bfloat16), jnp.int32)
            off += of

    return pl.pallas_call(
        body,
        grid=(steps,),
        in_specs=[pl.BlockSpec((BE, Da), lambda i: (i, 0)),
                  pl.BlockSpec((Da, W), lambda i: (0, 0))],
        out_specs=[pl.BlockSpec((g, BE // 2, _DW), lambda i: (0, i, 0))
                   for g in gs],
        out_shape=[jax.ShapeDtypeStruct((g, E // 2, _DW), jnp.int32)
                   for g in gs],
    )(edge_attr, wcat)


def _hmsg_call(feat, wtop, G):
    N, in_f = feat.shape
    out_f = wtop.shape[1]
    BN = 2000
    steps = N // BN

    def body(f_ref, w_ref, o_ref):
        res = jnp.dot(f_ref[...], w_ref[...], preferred_element_type=jnp.float32)
        for c in range(G):
            lo = c * _DW
            hi = min(out_f, lo + _DW)
            blk = res[:, lo:hi]
            if hi - lo < _DW:
                blk = jnp.concatenate(
                    [blk, jnp.zeros((BN, _DW - (hi - lo)), jnp.float32)], axis=1)
            o_ref[c] = blk

    return pl.pallas_call(
        body,
        grid=(steps,),
        in_specs=[pl.BlockSpec((BN, in_f), lambda i: (i, 0)),
                  pl.BlockSpec((in_f, out_f), lambda i: (0, 0))],
        out_specs=pl.BlockSpec((G, BN, _DW), lambda i: (0, i, 0)),
        out_shape=jax.ShapeDtypeStruct((G, N, _DW), jnp.float32),
    )(feat, wtop)


def _post_call(feat, agg, rdeg, wself, b_row, wcls_row, G, deg_col=None):
    N, in_f = feat.shape
    out_f = wself.shape[1]
    BN = 2000
    steps = N // BN
    first = rdeg is None

    def body(*refs):
        if first:
            f_ref, a_ref, w_ref, b_ref, wc_ref, fo_ref, rd_ref, s_ref = refs
        else:
            f_ref, a_ref, rd_in, w_ref, b_ref, wc_ref, fo_ref, s_ref = refs
        i = pl.program_id(0)
        if G == 1:
            asum = a_ref[0] + a_ref[1]
            agg_b = asum[:, :out_f]
        else:
            agg_b = jnp.concatenate([a_ref[0], a_ref[1]], axis=1)
        if first:
            deg = a_ref[0, :, deg_col:deg_col + 1] + a_ref[1, :, deg_col:deg_col + 1]
            rd = 1.0 / jnp.maximum(deg, 1.0)
            rd_ref[...] = rd
        else:
            rd = rd_in[...]
        f = jnp.maximum(
            jnp.dot(f_ref[...], w_ref[...], preferred_element_type=jnp.float32)
            + agg_b * rd + b_ref[...], 0.0)
        fo_ref[...] = f
        ps = jnp.sum(f * wc_ref[...]).reshape(1, 1)

        @pl.when(i == 0)
        def _():
            s_ref[...] = jnp.zeros((1, 1), jnp.float32)

        s_ref[...] += ps

    in_specs = [pl.BlockSpec((BN, in_f), lambda i: (i, 0)),
                pl.BlockSpec((_NC, BN, _DW), lambda i: (0, i, 0))]
    operands = [feat, agg]
    if not first:
        in_specs.append(pl.BlockSpec((BN, 1), lambda i: (i, 0)))
        operands.append(rdeg)
    in_specs += [pl.BlockSpec((in_f, out_f), lambda i: (0, 0)),
                 pl.BlockSpec((1, out_f), lambda i: (0, 0)),
                 pl.BlockSpec((1, out_f), lambda i: (0, 0))]
    operands += [wself, b_row, wcls_row]
    out_specs = [pl.BlockSpec((BN, out_f), lambda i: (i, 0))]
    out_shape = [jax.ShapeDtypeStruct((N, out_f), jnp.float32)]
    if first:
        out_specs.append(pl.BlockSpec((BN, 1), lambda i: (i, 0)))
        out_shape.append(jax.ShapeDtypeStruct((N, 1), jnp.float32))
    out_specs.append(pl.BlockSpec((1, 1), lambda i: (0, 0)))
    out_shape.append(jax.ShapeDtypeStruct((1, 1), jnp.float32))

    return pl.pallas_call(
        body,
        grid=(steps,),
        in_specs=in_specs,
        out_specs=out_specs,
        out_shape=out_shape,
    )(*operands)


def kernel(x, edge_index, edge_attr, params):
    N = x.shape[0]
    E = edge_index.shape[1]
    src = edge_index[0].astype(jnp.int32)
    dst = edge_index[1].astype(jnp.int32)

    in_fs = [p[1].shape[0] for p in params]
    out_fs = [p[0].shape[1] for p in params]
    gs = [-(-of // _DW) for of in out_fs]
    assert out_fs[0] < _DW, "layer-0 padding must have room for the degree column"
    wtops = [p[0][:f] for p, f in zip(params, in_fs)]
    wbots = [p[0][f:] for p, f in zip(params, in_fs)]

    eps = _eproj_call(edge_attr, wbots, gs)

    feat = x
    rdeg = None
    score = None
    for l, (W_msg, W_self, b, W_cls, b_cls) in enumerate(params):
        hm = _hmsg_call(feat, wtops[l], gs[l])
        nj = -(-(out_fs[l] + (1 if l == 0 else 0)) // 16) if gs[l] == 1 else _DW // 16
        agg = _edge_call(hm, eps[l], src, dst, N, E, gs[l], nj)
        outs = _post_call(feat, agg, rdeg, W_self, b.reshape(1, -1),
                          jnp.transpose(W_cls), gs[l],
                          deg_col=(out_fs[0] if l == 0 else None))
        if l == 0:
            feat, rdeg, s = outs
        else:
            feat, s = outs
        contrib = s / N + b_cls.reshape(1, 1)
        score = contrib if score is None else score + contrib
    return score
